# baseline (device time: 8600 ns/iter reference)
import jax
import jax.numpy as jnp
from jax import lax
from jax.experimental import pallas as pl
from jax.experimental.pallas import tpu as pltpu

N_SPLIT = 2


def kernel(x):
    m, n = x.shape
    half = n // 2
    rows = m // N_SPLIT

    def body(x_ref, out_ref, local_sem, send_sems, recv_sems):
        my_x = lax.axis_index("x")
        my_y = lax.axis_index("y")
        my_z = lax.axis_index("z")
        partner = (my_x, 1 - my_y, my_z)

        local = pltpu.make_async_copy(
            x_ref.at[:, pl.ds(my_y * half, half)],
            out_ref.at[pl.ds(my_y * m, m), :],
            local_sem,
        )
        local.start()

        barrier = pltpu.get_barrier_semaphore()
        pl.semaphore_signal(
            barrier, inc=1, device_id=partner,
            device_id_type=pl.DeviceIdType.MESH,
        )
        pl.semaphore_wait(barrier, 1)

        rdmas = []
        for s in range(N_SPLIT):
            rdma = pltpu.make_async_remote_copy(
                src_ref=x_ref.at[
                    pl.ds(s * rows, rows), pl.ds((1 - my_y) * half, half)
                ],
                dst_ref=out_ref.at[pl.ds(my_y * m + s * rows, rows), :],
                send_sem=send_sems.at[s],
                recv_sem=recv_sems.at[s],
                device_id=partner,
                device_id_type=pl.DeviceIdType.MESH,
            )
            rdma.start()
            rdmas.append(rdma)

        local.wait()
        for rdma in rdmas:
            rdma.wait()

    return pl.pallas_call(
        body,
        out_shape=jax.ShapeDtypeStruct((2 * m, half), x.dtype),
        in_specs=[pl.BlockSpec(memory_space=pltpu.VMEM)],
        out_specs=pl.BlockSpec(memory_space=pltpu.VMEM),
        scratch_shapes=[
            pltpu.SemaphoreType.DMA,
            pltpu.SemaphoreType.DMA((N_SPLIT,)),
            pltpu.SemaphoreType.DMA((N_SPLIT,)),
        ],
        compiler_params=pltpu.CompilerParams(collective_id=0),
    )(x)
